# Initial kernel scaffold; baseline (speedup 1.0000x reference)
import jax
import jax.numpy as jnp
from jax.experimental import pallas as pl


def kernel(user_emb, entity_emb, item_emb_cf, relation_weight, edge_index, edge_type, inter_row, inter_col):
    raise NotImplementedError("write your pallas kernel here")


# speedup vs baseline: 1.0266x; 1.0266x over previous
import jax
import jax.numpy as jnp
from jax.experimental import pallas as pl

N_USERS = 8000
N_ITEMS = 4000
N_ENTITIES = 10000
CH = 128
C = 1.0


def _norm(x):
    return jnp.clip(jnp.linalg.norm(x, axis=-1, keepdims=True), 1e-15, None)


def _project(x):
    n = _norm(x)
    maxn = 1.0 - 1e-3
    return jnp.where(n > maxn, x / n * maxn, x)


def _mobius_add(x, y):
    x2 = jnp.sum(x * x, -1, keepdims=True)
    y2 = jnp.sum(y * y, -1, keepdims=True)
    xy = jnp.sum(x * y, -1, keepdims=True)
    num = (1.0 + 2.0 * C * xy + C * y2) * x + (1.0 - C * x2) * y
    den = 1.0 + 2.0 * C * xy + C * C * x2 * y2
    return num / jnp.clip(den, 1e-15, None)


def _expmap0(u):
    un = _norm(u)
    return jnp.tanh(un) * u / un


def _lam(p):
    return 2.0 / jnp.clip(1.0 - C * jnp.sum(p * p, -1, keepdims=True), 1e-15, None)


def _expmap(u, p):
    un = _norm(u)
    second = jnp.tanh(_lam(p) * un / 2.0) * u / un
    return _mobius_add(p, second)


def _logmap(x, p):
    sub = _mobius_add(-p, x)
    sn = _norm(sub)
    return (2.0 / _lam(p)) * jnp.arctanh(jnp.clip(sn, 1e-15, 1.0 - 1e-7)) * sub / sn


def _scatter_mean(src, index, dim_size):
    s = jax.ops.segment_sum(src, index, num_segments=dim_size)
    cnt = jax.ops.segment_sum(jnp.ones((src.shape[0],), src.dtype), index, num_segments=dim_size)
    return s / jnp.clip(cnt, 1.0, None)[:, None]


def _l2n(x):
    return x / jnp.clip(jnp.linalg.norm(x, axis=-1, keepdims=True), 1e-12, None)


def _aggregate(entity_emb, user_emb, item_emb_cf, edge_index, edge_type, inter_row, inter_col, relation_weight):
    head = edge_index[0]
    tail = edge_index[1]
    rel = jnp.take(relation_weight, edge_type - 1, axis=0)
    head_emb = jnp.take(entity_emb, head, axis=0)
    tail_emb = jnp.take(entity_emb, tail, axis=0)
    hyper_head = _expmap0(head_emb)
    hyper_tail = _expmap(tail_emb, hyper_head)
    hyper_rel = _expmap(rel, hyper_head)
    res = _logmap(_project(_mobius_add(hyper_tail, hyper_rel)), hyper_head)
    entity_agg = _scatter_mean(res, head, entity_emb.shape[0])
    item_agg_cf = jax.ops.segment_sum(jnp.take(user_emb, inter_row, axis=0), inter_col, num_segments=N_ITEMS)
    item_fusion = item_emb_cf + entity_emb[:N_ITEMS]
    user_agg = jax.ops.segment_sum(jnp.take(item_fusion, inter_col, axis=0), inter_row, num_segments=N_USERS)
    return entity_agg, user_agg, item_agg_cf


def _dr_norm(user_emb, entity_emb, item_emb_cf):
    for i in range(3):
        entity_emb = _l2n(entity_emb)
        user_emb = _l2n(user_emb)
        item_emb_cf = _l2n(item_emb_cf)
        if i == 0:
            ea, ua, ia = entity_emb, user_emb, item_emb_cf
        else:
            ea = ea + entity_emb
            ua = ua + user_emb
            ia = ia + item_emb_cf
    return ea, ua, ia


def _add_kernel(a_ref, b_ref, o_ref):
    o_ref[...] = a_ref[...] + b_ref[...]


def _padd(a, b):
    return pl.pallas_call(
        _add_kernel,
        out_shape=jax.ShapeDtypeStruct(a.shape, a.dtype),
    )(a, b)


def kernel(user_emb, entity_emb, item_emb_cf, relation_weight, edge_index, edge_type, inter_row, inter_col):
    entity_res, user_res, item_res = entity_emb, user_emb, item_emb_cf
    ee, ue, ie = _aggregate(entity_emb, user_emb, item_emb_cf, edge_index, edge_type, inter_row, inter_col, relation_weight)
    ea, ua, ia = _dr_norm(ue, ee, ie)
    entity_res = _padd(entity_res, ea)
    user_res = _padd(user_res, ua)
    item_res = _padd(item_res, ia)
    for _ in range(3):
        ee, ue, ie = _aggregate(ea, ua, ia, edge_index, edge_type, inter_row, inter_col, relation_weight)
        ea, ua, ia = _dr_norm(ue, ee, ie)
    entity_res = _padd(entity_res, ea)
    user_res = _padd(user_res, ua)
    item_res = _padd(item_res, ia)
    return (entity_res, user_res, item_res)


# speedup vs baseline: 1.4228x; 1.3859x over previous
import functools

import jax
import jax.numpy as jnp
from jax import lax
from jax.experimental import pallas as pl
from jax.experimental.pallas import tpu as pltpu
from jax.experimental.pallas import tpu_sc as plsc

N_USERS = 8000
N_ITEMS = 4000
N_ENTITIES = 10000
N_EDGES = 320000
N_INTER = 160000
CH = 128

NE_PAD = 10240
NU_PAD = 8192
NI_PAD = 4096

NW = 32
EDGES_PER_W = N_EDGES // NW
SCA_S = 400
SCA_SUB = EDGES_PER_W // SCA_S
INTER_PER_W = N_INTER // NW
PUSH_S = 200
PUSH_SUB = INTER_PER_W // PUSH_S

_f32 = jnp.float32
_i32 = jnp.int32


def _rel_body(rw_ref, urp_ref, rnb_ref):
    r = rw_ref[...]
    n2 = jnp.sum(r * r, axis=1, keepdims=True)
    rn = jnp.clip(jnp.sqrt(n2), 1e-15, None)
    urp_ref[...] = r / rn
    rnb_ref[...] = jnp.broadcast_to(rn, r.shape)


def _rel_prep(rw):
    rwp = jnp.pad(rw, ((0, 16 - rw.shape[0]), (0, 0)))
    urp, rnb = pl.pallas_call(
        _rel_body,
        out_shape=(jax.ShapeDtypeStruct((16, CH), _f32),
                   jax.ShapeDtypeStruct((16, CH), _f32)),
    )(rwp)
    return urp, rnb[:, 0]


def _ent_body(e_ref, urpt_ref, u_ref, duur_ref, th_ref, hn_ref, nu2_ref):
    e = e_ref[...]
    n2 = jnp.sum(e * e, axis=1, keepdims=True)
    hn = jnp.clip(jnp.sqrt(n2), 1e-15, None)
    u = e / hn
    u_ref[...] = u
    duur_ref[...] = jnp.dot(u, urpt_ref[...], preferred_element_type=_f32)
    th_ref[...] = jnp.tanh(hn)
    hn_ref[...] = hn
    nu2_ref[...] = n2 / (hn * hn)


def _ent_prep(e, urpt):
    blk = 1000
    grid = N_ENTITIES // blk
    spec = pl.BlockSpec((blk, CH), lambda i: (i, 0))
    spec16 = pl.BlockSpec((blk, 16), lambda i: (i, 0))
    spec1 = pl.BlockSpec((blk, 1), lambda i: (i, 0))
    u, duur, th, hn, nu2 = pl.pallas_call(
        _ent_body,
        grid=(grid,),
        in_specs=[spec, pl.BlockSpec((CH, 16), lambda i: (0, 0))],
        out_specs=(spec, spec16, spec1, spec1, spec1),
        out_shape=(jax.ShapeDtypeStruct((N_ENTITIES, CH), _f32),
                   jax.ShapeDtypeStruct((N_ENTITIES, 16), _f32),
                   jax.ShapeDtypeStruct((N_ENTITIES, 1), _f32),
                   jax.ShapeDtypeStruct((N_ENTITIES, 1), _f32),
                   jax.ShapeDtypeStruct((N_ENTITIES, 1), _f32)),
    )(e, urpt)
    return (u, duur.reshape(-1), th.reshape(-1), hn.reshape(-1),
            nu2.reshape(-1))


def _sca_body(head, tail, ktyp, U, duurf, th1, hn1, nu21, rn16,
              o_uh, o_ut, o_g13, o_g23, o_thh, o_g11, o_hnt, o_g22, o_rnk,
              idxh, idxt, idxk, fidxh, fidxt, Uh, Ut,
              s_g13, s_g23, s_thh, s_g11, s_hnt, s_g22, s_rnk, sem):
    cid = lax.axis_index("c")
    sid = lax.axis_index("s")
    wid = cid * 16 + sid

    def sub(s, carry):
        base = wid * EDGES_PER_W + s * SCA_S
        sl_out = pl.ds(base, SCA_S)
        pltpu.sync_copy(head.at[sl_out], idxh)
        pltpu.sync_copy(tail.at[sl_out], idxt)
        pltpu.sync_copy(ktyp.at[sl_out], idxk)
        for g in range(SCA_S // 16):
            sl = pl.ds(g * 16, 16)
            kk = idxk[sl]
            fidxh[sl] = idxh[sl] * 16 + kk
            fidxt[sl] = idxt[sl] * 16 + kk
        pltpu.async_copy(U.at[idxh], Uh, sem).wait()
        pltpu.async_copy(U.at[idxt], Ut, sem).wait()
        pltpu.async_copy(duurf.at[fidxh], s_g13, sem).wait()
        pltpu.async_copy(duurf.at[fidxt], s_g23, sem).wait()
        pltpu.async_copy(th1.at[idxh], s_thh, sem).wait()
        pltpu.async_copy(nu21.at[idxh], s_g11, sem).wait()
        pltpu.async_copy(hn1.at[idxt], s_hnt, sem).wait()
        pltpu.async_copy(nu21.at[idxt], s_g22, sem).wait()
        pltpu.async_copy(rn16.at[idxk], s_rnk, sem).wait()
        pltpu.sync_copy(Uh, o_uh.at[sl_out])
        pltpu.sync_copy(Ut, o_ut.at[sl_out])
        pltpu.sync_copy(s_g13, o_g13.at[sl_out])
        pltpu.sync_copy(s_g23, o_g23.at[sl_out])
        pltpu.sync_copy(s_thh, o_thh.at[sl_out])
        pltpu.sync_copy(s_g11, o_g11.at[sl_out])
        pltpu.sync_copy(s_hnt, o_hnt.at[sl_out])
        pltpu.sync_copy(s_g22, o_g22.at[sl_out])
        pltpu.sync_copy(s_rnk, o_rnk.at[sl_out])
        return carry
    lax.fori_loop(0, SCA_SUB, sub, 0)


def _sca(head, tail, ktyp, U, duurf, th1, hn1, nu21, rn16):
    mesh = plsc.VectorSubcoreMesh(core_axis_name="c", subcore_axis_name="s")
    ev = jax.ShapeDtypeStruct((N_EDGES,), _f32)
    em = jax.ShapeDtypeStruct((N_EDGES, CH), _f32)
    fn = pl.kernel(
        _sca_body,
        mesh=mesh,
        out_type=(em, em) + tuple(ev for _ in range(7)),
        scratch_types=[
            pltpu.VMEM((SCA_S,), _i32), pltpu.VMEM((SCA_S,), _i32),
            pltpu.VMEM((SCA_S,), _i32), pltpu.VMEM((SCA_S,), _i32),
            pltpu.VMEM((SCA_S,), _i32),
            pltpu.VMEM((SCA_S, CH), _f32), pltpu.VMEM((SCA_S, CH), _f32),
        ] + [pltpu.VMEM((SCA_S,), _f32) for _ in range(7)] + [
            pltpu.SemaphoreType.DMA,
        ],
    )
    return fn(head, tail, ktyp, U, duurf, th1, hn1, nu21, rn16)


def _chain_body(uh_ref, ut_ref, g13_r, g23_r, thh_r, g11_r, hnt_r, g22_r,
                rnk_r, kt_r, urp_ref, msg_ref):
    uh = uh_ref[...]
    ut = ut_ref[...]
    g12 = jnp.sum(uh * ut, axis=1, keepdims=True)
    g13 = g13_r[...]
    g23 = g23_r[...]
    th = thh_r[...]
    g11 = g11_r[...]
    hnt = hnt_r[...]
    g22 = g22_r[...]
    rnk = rnk_r[...]

    p2 = th * th * g11
    lam = 2.0 / jnp.clip(1.0 - p2, 1e-15, None)
    a_t = jnp.tanh(lam * hnt * 0.5)
    a_r = jnp.tanh(lam * rnk * 0.5)

    xy1 = th * a_t * g12
    y21 = a_t * a_t * g22
    den1 = jnp.clip(1.0 + 2.0 * xy1 + p2 * y21, 1e-15, None)
    c1 = (1.0 + 2.0 * xy1 + y21) * th / den1
    c2 = (1.0 - p2) * a_t / den1

    xy2 = th * a_r * g13
    y22 = a_r * a_r
    den2 = jnp.clip(1.0 + 2.0 * xy2 + p2 * y22, 1e-15, None)
    d1 = (1.0 + 2.0 * xy2 + y22) * th / den2
    d3 = (1.0 - p2) * a_r / den2

    def quad(v1, v2, v3, w1, w2, w3):
        return (v1 * w1 * g11 + v2 * w2 * g22 + v3 * w3
                + (v1 * w2 + v2 * w1) * g12 + (v1 * w3 + v3 * w1) * g13
                + (v2 * w3 + v3 * w2) * g23)

    xx = quad(c1, c2, 0.0, c1, c2, 0.0)
    yy = quad(d1, 0.0, d3, d1, 0.0, d3)
    xy = quad(c1, c2, 0.0, d1, 0.0, d3)
    den = jnp.clip(1.0 + 2.0 * xy + xx * yy, 1e-15, None)
    A = (1.0 + 2.0 * xy + yy) / den
    B = (1.0 - xx) / den
    e1 = A * c1 + B * d1
    e2 = A * c2
    e3 = B * d3

    pn2 = quad(e1, e2, e3, e1, e2, e3)
    n = jnp.clip(jnp.sqrt(jnp.maximum(pn2, 0.0)), 1e-15, None)
    maxn = 1.0 - 1e-3
    scl = jnp.where(n > maxn, maxn / n, 1.0)
    e1 = e1 * scl
    e2 = e2 * scl
    e3 = e3 * scl
    y2p = pn2 * scl * scl

    xys = -th * (e1 * g11 + e2 * g12 + e3 * g13)
    dens = jnp.clip(1.0 + 2.0 * xys + p2 * y2p, 1e-15, None)
    As = (1.0 + 2.0 * xys + y2p) / dens
    Bs = (1.0 - p2) / dens
    f1 = -As * th + Bs * e1
    f2 = Bs * e2
    f3 = Bs * e3

    sn2 = quad(f1, f2, f3, f1, f2, f3)
    sn = jnp.clip(jnp.sqrt(jnp.maximum(sn2, 0.0)), 1e-15, None)
    snc = jnp.clip(sn, 1e-15, 1.0 - 1e-7)
    atanh = 0.5 * jnp.log((1.0 + snc) / (1.0 - snc))
    fac = (2.0 / lam) * atanh / sn
    al = fac * f1
    be = fac * f2
    ga = fac * f3

    lane = lax.broadcasted_iota(_i32, (1, 16), 1)
    oh = jnp.where(lane == kt_r[...], ga, 0.0)
    msg_ref[...] = (al * uh + be * ut
                    + jnp.dot(oh, urp_ref[...], preferred_element_type=_f32))


def _edge_chain(uh, ut, g13, g23, thh, g11, hnt, g22, rnk, ktyp, urp):
    blk = 2000
    grid = N_EDGES // blk
    spec = pl.BlockSpec((blk, CH), lambda i: (i, 0))
    spec1 = pl.BlockSpec((blk, 1), lambda i: (i, 0))
    scal = [x.reshape(N_EDGES, 1) for x in (g13, g23, thh, g11, hnt, g22, rnk)]
    kt2 = ktyp.reshape(N_EDGES, 1)
    msg = pl.pallas_call(
        _chain_body,
        grid=(grid,),
        in_specs=[spec, spec] + [spec1] * 8
                 + [pl.BlockSpec((16, CH), lambda i: (0, 0))],
        out_specs=spec,
        out_shape=jax.ShapeDtypeStruct((N_EDGES, CH), _f32),
    )(uh, ut, *scal, kt2, urp)
    return msg


EHALF = 5120
EACC = 5376
GROW = EHALF
SCB_PER_SC_W = N_EDGES // 16
SCB_SUB = SCB_PER_SC_W // SCA_S


def _scb_body(head, msg, sb, idxh, idx2, rows, zb, acc_sp, sem):
    cid = lax.axis_index("c")
    sid = lax.axis_index("s")
    off = cid * EHALF
    stripe = EACC // 16

    for r in range(16):
        for c in range(8):
            zb[r, pl.ds(c * 16, 16)] = jnp.zeros((16,), _f32)

    def zfill(z, carry):
        pltpu.sync_copy(zb, acc_sp.at[pl.ds(sid * stripe + z * 16, 16)])
        return carry
    lax.fori_loop(0, stripe // 16, zfill, 0)
    plsc.subcore_barrier()

    def sub(s, carry):
        base = sid * SCB_PER_SC_W + s * SCA_S
        pltpu.sync_copy(head.at[pl.ds(base, SCA_S)], idxh)
        pltpu.sync_copy(msg.at[pl.ds(base, SCA_S)], rows)
        for g in range(SCA_S // 16):
            sl = pl.ds(g * 16, 16)
            d = idxh[sl] - off
            ok = (d >= 0) & (d < EHALF)
            idx2[sl] = jnp.where(ok, d, GROW)
        pltpu.sync_copy(rows, acc_sp.at[idx2], add=True)
        return carry
    lax.fori_loop(0, SCB_SUB, sub, 0)
    plsc.subcore_barrier()

    pltpu.sync_copy(acc_sp.at[pl.ds(sid * stripe, stripe)],
                    sb.at[pl.ds(cid * EACC + sid * stripe, stripe)])


def _scb(head, msg):
    mesh = plsc.VectorSubcoreMesh(core_axis_name="c", subcore_axis_name="s")
    fn = pl.kernel(
        _scb_body,
        mesh=mesh,
        out_type=jax.ShapeDtypeStruct((2 * EACC, CH), _f32),
        scratch_types=[
            pltpu.VMEM((SCA_S,), _i32), pltpu.VMEM((SCA_S,), _i32),
            pltpu.VMEM((SCA_S, CH), _f32),
            pltpu.VMEM((16, CH), _f32),
            pltpu.VMEM_SHARED((EACC, CH), _f32),
            pltpu.SemaphoreType.DMA,
        ],
    )
    return fn(head, msg)


_CNT_S = 80


def _cnt_body(head, cc, idxh, idx2, ones, zb, acc_sp, sem):
    cid = lax.axis_index("c")
    sid = lax.axis_index("s")
    off = cid * EHALF
    stripe = EACC // 16

    for r in range(16):
        for c in range(8):
            zb[r, pl.ds(c * 16, 16)] = jnp.zeros((16,), _f32)
    for r in range(_CNT_S):
        for c in range(8):
            ones[r, pl.ds(c * 16, 16)] = jnp.ones((16,), _f32)

    def zfill(z, carry):
        pltpu.sync_copy(zb, acc_sp.at[pl.ds(sid * stripe + z * 16, 16)])
        return carry
    lax.fori_loop(0, stripe // 16, zfill, 0)
    plsc.subcore_barrier()

    def sub(s, carry):
        base = sid * SCB_PER_SC_W + s * _CNT_S
        pltpu.sync_copy(head.at[pl.ds(base, _CNT_S)], idxh)
        for g in range(_CNT_S // 16):
            sl = pl.ds(g * 16, 16)
            d = idxh[sl] - off
            ok = (d >= 0) & (d < EHALF)
            idx2[sl] = jnp.where(ok, d, GROW)
        pltpu.sync_copy(ones, acc_sp.at[idx2], add=True)
        return carry
    lax.fori_loop(0, SCB_PER_SC_W // _CNT_S, sub, 0)
    plsc.subcore_barrier()

    pltpu.sync_copy(acc_sp.at[pl.ds(sid * stripe, stripe)],
                    cc.at[pl.ds(cid * EACC + sid * stripe, stripe)])


def _count(head):
    mesh = plsc.VectorSubcoreMesh(core_axis_name="c", subcore_axis_name="s")
    fn = pl.kernel(
        _cnt_body,
        mesh=mesh,
        out_type=jax.ShapeDtypeStruct((2 * EACC, CH), _f32),
        scratch_types=[
            pltpu.VMEM((_CNT_S,), _i32), pltpu.VMEM((_CNT_S,), _i32),
            pltpu.VMEM((_CNT_S, CH), _f32),
            pltpu.VMEM((16, CH), _f32),
            pltpu.VMEM_SHARED((EACC, CH), _f32),
            pltpu.SemaphoreType.DMA,
        ],
    )
    return fn(head)


def _make_push(n_dst_pad):
    stripe = n_dst_pad // 16

    def body(table, src_idx, dst_idx, po,
             isrc, idst, rows, zb, acc_sp, sem):
        cid = lax.axis_index("c")
        sid = lax.axis_index("s")
        wid = cid * 16 + sid

        for r in range(16):
            for c in range(8):
                zb[r, pl.ds(c * 16, 16)] = jnp.zeros((16,), _f32)

        def zfill(z, carry):
            pltpu.sync_copy(zb, acc_sp.at[pl.ds(sid * stripe + z * 16, 16)])
            return carry
        lax.fori_loop(0, stripe // 16, zfill, 0)
        plsc.subcore_barrier()

        def sub(s, carry):
            base = wid * INTER_PER_W + s * PUSH_S
            pltpu.sync_copy(src_idx.at[pl.ds(base, PUSH_S)], isrc)
            pltpu.sync_copy(dst_idx.at[pl.ds(base, PUSH_S)], idst)
            pltpu.async_copy(table.at[isrc], rows, sem).wait()
            pltpu.sync_copy(rows, acc_sp.at[idst], add=True)
            return carry
        lax.fori_loop(0, PUSH_SUB, sub, 0)
        plsc.subcore_barrier()

        pltpu.sync_copy(acc_sp.at[pl.ds(sid * stripe, stripe)],
                        po.at[pl.ds(cid * n_dst_pad + sid * stripe, stripe)])

    mesh = plsc.VectorSubcoreMesh(core_axis_name="c", subcore_axis_name="s")
    fn = pl.kernel(
        body,
        mesh=mesh,
        out_type=jax.ShapeDtypeStruct((2 * n_dst_pad, CH), _f32),
        scratch_types=[
            pltpu.VMEM((PUSH_S,), _i32), pltpu.VMEM((PUSH_S,), _i32),
            pltpu.VMEM((PUSH_S, CH), _f32),
            pltpu.VMEM((16, CH), _f32),
            pltpu.VMEM_SHARED((n_dst_pad, CH), _f32),
            pltpu.SemaphoreType.DMA,
        ],
    )
    return fn


_make_push = functools.cache(_make_push)


def _push_item(table, src_idx, dst_idx):
    return _make_push(NI_PAD)(table, src_idx, dst_idx)


def _push_user(table, src_idx, dst_idx):
    return _make_push(NU_PAD)(table, src_idx, dst_idx)


def _dr3(x, eps=1e-12):
    out = None
    for _ in range(3):
        n = jnp.clip(jnp.sqrt(jnp.sum(x * x, axis=1, keepdims=True)), eps, None)
        x = x / n
        out = x if out is None else out + x
    return out


def _fin_dense_body(p0_ref, p1_ref, o_ref):
    o_ref[...] = _dr3(p0_ref[...] + p1_ref[...])


def _fin_dense(p0, p1, n):
    blk = 1000
    grid = n // blk
    spec = pl.BlockSpec((blk, CH), lambda i: (i, 0))
    return pl.pallas_call(
        _fin_dense_body,
        grid=(grid,),
        in_specs=[spec, spec],
        out_specs=spec,
        out_shape=jax.ShapeDtypeStruct((n, CH), _f32),
    )(p0, p1)


def _fin_ent_body(sb_ref, c_ref, o_ref):
    cnt = jnp.clip(c_ref[...], 1.0, None)
    o_ref[...] = _dr3(sb_ref[...] / cnt)


def _fin_entity(sbA, sbB, cA, cB):
    blk = 1024
    grid = EHALF // blk
    spec = pl.BlockSpec((blk, CH), lambda i: (i, 0))
    halves = []
    for sb, c in ((sbA, cA), (sbB, cB)):
        halves.append(pl.pallas_call(
            _fin_ent_body,
            grid=(grid,),
            in_specs=[spec, spec],
            out_specs=spec,
            out_shape=jax.ShapeDtypeStruct((EHALF, CH), _f32),
        )(sb, c))
    return jnp.concatenate(halves, axis=0)[:N_ENTITIES]


def _add_body(a_ref, b_ref, o_ref):
    o_ref[...] = a_ref[...] + b_ref[...]


def _padd(a, b):
    return pl.pallas_call(
        _add_body,
        out_shape=jax.ShapeDtypeStruct(a.shape, a.dtype),
    )(a, b)


def kernel(user_emb, entity_emb, item_emb_cf, relation_weight, edge_index,
           edge_type, inter_row, inter_col):
    head = edge_index[0]
    tail = edge_index[1]
    ktyp = edge_type - 1
    urp, rn16 = _rel_prep(relation_weight)
    urpt = urp.T
    cnt = _count(head)
    cA, cB = cnt[:EACC], cnt[EACC:]

    er, ur_, ir = entity_emb, user_emb, item_emb_cf
    ea, ua, ia = entity_emb, user_emb, item_emb_cf
    for rnd in range(4):
        U, duurf, th1, hn1, nu21 = _ent_prep(ea, urpt)
        sc = _sca(head, tail, ktyp, U, duurf, th1, hn1, nu21, rn16)
        msg = _edge_chain(*sc, ktyp, urp)
        sb = _scb(head, msg)
        ip = _push_item(ua, inter_row, inter_col)
        fusion = _padd(ia, ea[:N_ITEMS])
        up = _push_user(fusion, inter_col, inter_row)
        ea = _fin_entity(sb[:EACC], sb[EACC:], cA, cB)
        ua = _fin_dense(up[:NU_PAD], up[NU_PAD:], N_USERS)
        ia = _fin_dense(ip[:NI_PAD], ip[NI_PAD:], N_ITEMS)
        if rnd == 0 or rnd == 3:
            er = _padd(er, ea)
            ur_ = _padd(ur_, ua)
            ir = _padd(ir, ia)
    return (er, ur_, ir)


# speedup vs baseline: 1.4297x; 1.0049x over previous
import functools

import jax
import jax.numpy as jnp
from jax import lax
from jax.experimental import pallas as pl
from jax.experimental.pallas import tpu as pltpu
from jax.experimental.pallas import tpu_sc as plsc

N_USERS = 8000
N_ITEMS = 4000
N_ENTITIES = 10000
N_EDGES = 320000
N_INTER = 160000
CH = 128

NE_PAD = 10240
NU_PAD = 8192
NI_PAD = 4096

NW = 32
EDGES_PER_W = N_EDGES // NW
SCA_S = 400
SCA_SUB = EDGES_PER_W // SCA_S
INTER_PER_W = N_INTER // NW
PUSH_S = 200
PUSH_SUB = INTER_PER_W // PUSH_S

_f32 = jnp.float32
_i32 = jnp.int32


def _rel_body(rw_ref, urp_ref, rnb_ref):
    r = rw_ref[...]
    n2 = jnp.sum(r * r, axis=1, keepdims=True)
    rn = jnp.clip(jnp.sqrt(n2), 1e-15, None)
    urp_ref[...] = r / rn
    rnb_ref[...] = jnp.broadcast_to(rn, r.shape)


def _rel_prep(rw):
    rwp = jnp.pad(rw, ((0, 16 - rw.shape[0]), (0, 0)))
    urp, rnb = pl.pallas_call(
        _rel_body,
        out_shape=(jax.ShapeDtypeStruct((16, CH), _f32),
                   jax.ShapeDtypeStruct((16, CH), _f32)),
    )(rwp)
    return urp, rnb[:, 0]


def _ent_body(e_ref, urpt_ref, u_ref, duur_ref, th_ref, hn_ref, nu2_ref):
    e = e_ref[...]
    n2 = jnp.sum(e * e, axis=1, keepdims=True)
    hn = jnp.clip(jnp.sqrt(n2), 1e-15, None)
    u = e / hn
    u_ref[...] = u
    duur_ref[...] = jnp.dot(u, urpt_ref[...], preferred_element_type=_f32)
    th_ref[...] = jnp.tanh(hn)
    hn_ref[...] = hn
    nu2_ref[...] = n2 / (hn * hn)


def _ent_prep(e, urpt):
    blk = 1000
    grid = N_ENTITIES // blk
    spec = pl.BlockSpec((blk, CH), lambda i: (i, 0))
    spec16 = pl.BlockSpec((blk, 16), lambda i: (i, 0))
    spec1 = pl.BlockSpec((blk, 1), lambda i: (i, 0))
    u, duur, th, hn, nu2 = pl.pallas_call(
        _ent_body,
        grid=(grid,),
        in_specs=[spec, pl.BlockSpec((CH, 16), lambda i: (0, 0))],
        out_specs=(spec, spec16, spec1, spec1, spec1),
        out_shape=(jax.ShapeDtypeStruct((N_ENTITIES, CH), _f32),
                   jax.ShapeDtypeStruct((N_ENTITIES, 16), _f32),
                   jax.ShapeDtypeStruct((N_ENTITIES, 1), _f32),
                   jax.ShapeDtypeStruct((N_ENTITIES, 1), _f32),
                   jax.ShapeDtypeStruct((N_ENTITIES, 1), _f32)),
    )(e, urpt)
    return (u, duur.reshape(-1), th.reshape(-1), hn.reshape(-1),
            nu2.reshape(-1))


def _sca_body(head, tail, ktyp, U, duurf, th1, hn1, nu21, rn16,
              o_uh, o_ut, o_g13, o_g23, o_thh, o_g11, o_hnt, o_g22, o_rnk,
              idxh, idxt, idxk, fidxh, fidxt, Uh, Ut,
              s_g13, s_g23, s_thh, s_g11, s_hnt, s_g22, s_rnk, sem):
    cid = lax.axis_index("c")
    sid = lax.axis_index("s")
    wid = cid * 16 + sid

    def sub(s, carry):
        base = wid * EDGES_PER_W + s * SCA_S
        sl_out = pl.ds(base, SCA_S)
        c1 = pltpu.async_copy(head.at[sl_out], idxh, sem)
        c2 = pltpu.async_copy(tail.at[sl_out], idxt, sem)
        c3 = pltpu.async_copy(ktyp.at[sl_out], idxk, sem)
        c1.wait()
        c2.wait()
        c3.wait()
        for g in range(SCA_S // 16):
            sl = pl.ds(g * 16, 16)
            kk = idxk[sl]
            fidxh[sl] = idxh[sl] * 16 + kk
            fidxt[sl] = idxt[sl] * 16 + kk
        gs = [
            pltpu.async_copy(U.at[idxh], Uh, sem),
            pltpu.async_copy(U.at[idxt], Ut, sem),
            pltpu.async_copy(duurf.at[fidxh], s_g13, sem),
            pltpu.async_copy(duurf.at[fidxt], s_g23, sem),
            pltpu.async_copy(th1.at[idxh], s_thh, sem),
            pltpu.async_copy(nu21.at[idxh], s_g11, sem),
            pltpu.async_copy(hn1.at[idxt], s_hnt, sem),
            pltpu.async_copy(nu21.at[idxt], s_g22, sem),
            pltpu.async_copy(rn16.at[idxk], s_rnk, sem),
        ]
        for g_ in gs:
            g_.wait()
        os_ = [
            pltpu.async_copy(Uh, o_uh.at[sl_out], sem),
            pltpu.async_copy(Ut, o_ut.at[sl_out], sem),
            pltpu.async_copy(s_g13, o_g13.at[sl_out], sem),
            pltpu.async_copy(s_g23, o_g23.at[sl_out], sem),
            pltpu.async_copy(s_thh, o_thh.at[sl_out], sem),
            pltpu.async_copy(s_g11, o_g11.at[sl_out], sem),
            pltpu.async_copy(s_hnt, o_hnt.at[sl_out], sem),
            pltpu.async_copy(s_g22, o_g22.at[sl_out], sem),
            pltpu.async_copy(s_rnk, o_rnk.at[sl_out], sem),
        ]
        for o_ in os_:
            o_.wait()
        return carry
    lax.fori_loop(0, SCA_SUB, sub, 0)


def _sca(head, tail, ktyp, U, duurf, th1, hn1, nu21, rn16):
    mesh = plsc.VectorSubcoreMesh(core_axis_name="c", subcore_axis_name="s")
    ev = jax.ShapeDtypeStruct((N_EDGES,), _f32)
    em = jax.ShapeDtypeStruct((N_EDGES, CH), _f32)
    fn = pl.kernel(
        _sca_body,
        mesh=mesh,
        out_type=(em, em) + tuple(ev for _ in range(7)),
        scratch_types=[
            pltpu.VMEM((SCA_S,), _i32), pltpu.VMEM((SCA_S,), _i32),
            pltpu.VMEM((SCA_S,), _i32), pltpu.VMEM((SCA_S,), _i32),
            pltpu.VMEM((SCA_S,), _i32),
            pltpu.VMEM((SCA_S, CH), _f32), pltpu.VMEM((SCA_S, CH), _f32),
        ] + [pltpu.VMEM((SCA_S,), _f32) for _ in range(7)] + [
            pltpu.SemaphoreType.DMA,
        ],
    )
    return fn(head, tail, ktyp, U, duurf, th1, hn1, nu21, rn16)


def _chain_body(uh_ref, ut_ref, g13_r, g23_r, thh_r, g11_r, hnt_r, g22_r,
                rnk_r, kt_r, urp_ref, msg_ref):
    uh = uh_ref[...]
    ut = ut_ref[...]
    g12 = jnp.sum(uh * ut, axis=1, keepdims=True)
    g13 = g13_r[...]
    g23 = g23_r[...]
    th = thh_r[...]
    g11 = g11_r[...]
    hnt = hnt_r[...]
    g22 = g22_r[...]
    rnk = rnk_r[...]
    lane = lax.broadcasted_iota(_i32, (1, 16), 1)
    ohk = lane == kt_r[...]

    p2 = th * th * g11
    lam = 2.0 / jnp.clip(1.0 - p2, 1e-15, None)
    a_t = jnp.tanh(lam * hnt * 0.5)
    a_r = jnp.tanh(lam * rnk * 0.5)

    xy1 = th * a_t * g12
    y21 = a_t * a_t * g22
    den1 = jnp.clip(1.0 + 2.0 * xy1 + p2 * y21, 1e-15, None)
    c1 = (1.0 + 2.0 * xy1 + y21) * th / den1
    c2 = (1.0 - p2) * a_t / den1

    xy2 = th * a_r * g13
    y22 = a_r * a_r
    den2 = jnp.clip(1.0 + 2.0 * xy2 + p2 * y22, 1e-15, None)
    d1 = (1.0 + 2.0 * xy2 + y22) * th / den2
    d3 = (1.0 - p2) * a_r / den2

    def quad(v1, v2, v3, w1, w2, w3):
        return (v1 * w1 * g11 + v2 * w2 * g22 + v3 * w3
                + (v1 * w2 + v2 * w1) * g12 + (v1 * w3 + v3 * w1) * g13
                + (v2 * w3 + v3 * w2) * g23)

    xx = quad(c1, c2, 0.0, c1, c2, 0.0)
    yy = quad(d1, 0.0, d3, d1, 0.0, d3)
    xy = quad(c1, c2, 0.0, d1, 0.0, d3)
    den = jnp.clip(1.0 + 2.0 * xy + xx * yy, 1e-15, None)
    A = (1.0 + 2.0 * xy + yy) / den
    B = (1.0 - xx) / den
    e1 = A * c1 + B * d1
    e2 = A * c2
    e3 = B * d3

    pn2 = quad(e1, e2, e3, e1, e2, e3)
    n = jnp.clip(jnp.sqrt(jnp.maximum(pn2, 0.0)), 1e-15, None)
    maxn = 1.0 - 1e-3
    scl = jnp.where(n > maxn, maxn / n, 1.0)
    e1 = e1 * scl
    e2 = e2 * scl
    e3 = e3 * scl
    y2p = pn2 * scl * scl

    xys = -th * (e1 * g11 + e2 * g12 + e3 * g13)
    dens = jnp.clip(1.0 + 2.0 * xys + p2 * y2p, 1e-15, None)
    As = (1.0 + 2.0 * xys + y2p) / dens
    Bs = (1.0 - p2) / dens
    f1 = -As * th + Bs * e1
    f2 = Bs * e2
    f3 = Bs * e3

    sn2 = quad(f1, f2, f3, f1, f2, f3)
    sn = jnp.clip(jnp.sqrt(jnp.maximum(sn2, 0.0)), 1e-15, None)
    snc = jnp.clip(sn, 1e-15, 1.0 - 1e-7)
    atanh = 0.5 * jnp.log((1.0 + snc) / (1.0 - snc))
    fac = (2.0 / lam) * atanh / sn
    al = fac * f1
    be = fac * f2
    ga = fac * f3

    oh = jnp.where(ohk, ga, 0.0)
    msg_ref[...] = (al * uh + be * ut
                    + jnp.dot(oh, urp_ref[...], preferred_element_type=_f32))


def _edge_chain(uh, ut, g13, g23, thh, g11, hnt, g22, rnk, ktyp, urp):
    blk = 2000
    grid = N_EDGES // blk
    spec = pl.BlockSpec((blk, CH), lambda i: (i, 0))
    spec1 = pl.BlockSpec((blk, 1), lambda i: (i, 0))
    scal = [x.reshape(N_EDGES, 1) for x in (g13, g23, thh, g11, hnt, g22, rnk)]
    kt2 = ktyp.reshape(N_EDGES, 1)
    msg = pl.pallas_call(
        _chain_body,
        grid=(grid,),
        in_specs=[spec, spec] + [spec1] * 8
                 + [pl.BlockSpec((16, CH), lambda i: (0, 0))],
        out_specs=spec,
        out_shape=jax.ShapeDtypeStruct((N_EDGES, CH), _f32),
    )(uh, ut, *scal, kt2, urp)
    return msg


EHALF = 5120
EACC = 5376
GROW = EHALF
SCB_PER_SC_W = N_EDGES // 16
SCB_SUB = SCB_PER_SC_W // SCA_S


def _scb_body(head, msg, sb, idxh, idx2, rows, zb, acc_sp, sem):
    cid = lax.axis_index("c")
    sid = lax.axis_index("s")
    off = cid * EHALF
    stripe = EACC // 16

    for r in range(16):
        for c in range(8):
            zb[r, pl.ds(c * 16, 16)] = jnp.zeros((16,), _f32)

    def zfill(z, carry):
        pltpu.sync_copy(zb, acc_sp.at[pl.ds(sid * stripe + z * 16, 16)])
        return carry
    lax.fori_loop(0, stripe // 16, zfill, 0)
    plsc.subcore_barrier()

    def sub(s, carry):
        base = sid * SCB_PER_SC_W + s * SCA_S
        c1 = pltpu.async_copy(head.at[pl.ds(base, SCA_S)], idxh, sem)
        c2 = pltpu.async_copy(msg.at[pl.ds(base, SCA_S)], rows, sem)
        c1.wait()
        for g in range(SCA_S // 16):
            sl = pl.ds(g * 16, 16)
            d = idxh[sl] - off
            ok = (d >= 0) & (d < EHALF)
            idx2[sl] = jnp.where(ok, d, GROW)
        c2.wait()
        pltpu.sync_copy(rows, acc_sp.at[idx2], add=True)
        return carry
    lax.fori_loop(0, SCB_SUB, sub, 0)
    plsc.subcore_barrier()

    pltpu.sync_copy(acc_sp.at[pl.ds(sid * stripe, stripe)],
                    sb.at[pl.ds(cid * EACC + sid * stripe, stripe)])


def _scb(head, msg):
    mesh = plsc.VectorSubcoreMesh(core_axis_name="c", subcore_axis_name="s")
    fn = pl.kernel(
        _scb_body,
        mesh=mesh,
        out_type=jax.ShapeDtypeStruct((2 * EACC, CH), _f32),
        scratch_types=[
            pltpu.VMEM((SCA_S,), _i32), pltpu.VMEM((SCA_S,), _i32),
            pltpu.VMEM((SCA_S, CH), _f32),
            pltpu.VMEM((16, CH), _f32),
            pltpu.VMEM_SHARED((EACC, CH), _f32),
            pltpu.SemaphoreType.DMA,
        ],
    )
    return fn(head, msg)


_CNT_S = 80


def _cnt_body(head, cc, idxh, idx2, ones, zb, acc_sp, sem):
    cid = lax.axis_index("c")
    sid = lax.axis_index("s")
    off = cid * EHALF
    stripe = EACC // 16

    for r in range(16):
        for c in range(8):
            zb[r, pl.ds(c * 16, 16)] = jnp.zeros((16,), _f32)
    for r in range(_CNT_S):
        for c in range(8):
            ones[r, pl.ds(c * 16, 16)] = jnp.ones((16,), _f32)

    def zfill(z, carry):
        pltpu.sync_copy(zb, acc_sp.at[pl.ds(sid * stripe + z * 16, 16)])
        return carry
    lax.fori_loop(0, stripe // 16, zfill, 0)
    plsc.subcore_barrier()

    def sub(s, carry):
        base = sid * SCB_PER_SC_W + s * _CNT_S
        pltpu.sync_copy(head.at[pl.ds(base, _CNT_S)], idxh)
        for g in range(_CNT_S // 16):
            sl = pl.ds(g * 16, 16)
            d = idxh[sl] - off
            ok = (d >= 0) & (d < EHALF)
            idx2[sl] = jnp.where(ok, d, GROW)
        pltpu.sync_copy(ones, acc_sp.at[idx2], add=True)
        return carry
    lax.fori_loop(0, SCB_PER_SC_W // _CNT_S, sub, 0)
    plsc.subcore_barrier()

    pltpu.sync_copy(acc_sp.at[pl.ds(sid * stripe, stripe)],
                    cc.at[pl.ds(cid * EACC + sid * stripe, stripe)])


def _count(head):
    mesh = plsc.VectorSubcoreMesh(core_axis_name="c", subcore_axis_name="s")
    fn = pl.kernel(
        _cnt_body,
        mesh=mesh,
        out_type=jax.ShapeDtypeStruct((2 * EACC, CH), _f32),
        scratch_types=[
            pltpu.VMEM((_CNT_S,), _i32), pltpu.VMEM((_CNT_S,), _i32),
            pltpu.VMEM((_CNT_S, CH), _f32),
            pltpu.VMEM((16, CH), _f32),
            pltpu.VMEM_SHARED((EACC, CH), _f32),
            pltpu.SemaphoreType.DMA,
        ],
    )
    return fn(head)


def _make_push(n_dst_pad):
    stripe = n_dst_pad // 16

    def body(table, src_idx, dst_idx, po,
             isrc, idst, rows, zb, acc_sp, sem):
        cid = lax.axis_index("c")
        sid = lax.axis_index("s")
        wid = cid * 16 + sid

        for r in range(16):
            for c in range(8):
                zb[r, pl.ds(c * 16, 16)] = jnp.zeros((16,), _f32)

        def zfill(z, carry):
            pltpu.sync_copy(zb, acc_sp.at[pl.ds(sid * stripe + z * 16, 16)])
            return carry
        lax.fori_loop(0, stripe // 16, zfill, 0)
        plsc.subcore_barrier()

        def sub(s, carry):
            base = wid * INTER_PER_W + s * PUSH_S
            c1 = pltpu.async_copy(src_idx.at[pl.ds(base, PUSH_S)], isrc, sem)
            c2 = pltpu.async_copy(dst_idx.at[pl.ds(base, PUSH_S)], idst, sem)
            c1.wait()
            c2.wait()
            pltpu.async_copy(table.at[isrc], rows, sem).wait()
            pltpu.sync_copy(rows, acc_sp.at[idst], add=True)
            return carry
        lax.fori_loop(0, PUSH_SUB, sub, 0)
        plsc.subcore_barrier()

        pltpu.sync_copy(acc_sp.at[pl.ds(sid * stripe, stripe)],
                        po.at[pl.ds(cid * n_dst_pad + sid * stripe, stripe)])

    mesh = plsc.VectorSubcoreMesh(core_axis_name="c", subcore_axis_name="s")
    fn = pl.kernel(
        body,
        mesh=mesh,
        out_type=jax.ShapeDtypeStruct((2 * n_dst_pad, CH), _f32),
        scratch_types=[
            pltpu.VMEM((PUSH_S,), _i32), pltpu.VMEM((PUSH_S,), _i32),
            pltpu.VMEM((PUSH_S, CH), _f32),
            pltpu.VMEM((16, CH), _f32),
            pltpu.VMEM_SHARED((n_dst_pad, CH), _f32),
            pltpu.SemaphoreType.DMA,
        ],
    )
    return fn


_make_push = functools.cache(_make_push)


def _push_item(table, src_idx, dst_idx):
    return _make_push(NI_PAD)(table, src_idx, dst_idx)


def _push_user(table, src_idx, dst_idx):
    return _make_push(NU_PAD)(table, src_idx, dst_idx)


def _dr3(x, eps=1e-12):
    out = None
    for _ in range(3):
        n = jnp.clip(jnp.sqrt(jnp.sum(x * x, axis=1, keepdims=True)), eps, None)
        x = x / n
        out = x if out is None else out + x
    return out


def _fin_dense_body(p0_ref, p1_ref, o_ref):
    o_ref[...] = _dr3(p0_ref[...] + p1_ref[...])


def _fin_dense(p0, p1, n):
    blk = 1000
    grid = n // blk
    spec = pl.BlockSpec((blk, CH), lambda i: (i, 0))
    return pl.pallas_call(
        _fin_dense_body,
        grid=(grid,),
        in_specs=[spec, spec],
        out_specs=spec,
        out_shape=jax.ShapeDtypeStruct((n, CH), _f32),
    )(p0, p1)


def _fin_ent_body(sb_ref, c_ref, o_ref):
    cnt = jnp.clip(c_ref[...], 1.0, None)
    o_ref[...] = _dr3(sb_ref[...] / cnt)


def _fin_entity(sbA, sbB, cA, cB):
    blk = 1024
    grid = EHALF // blk
    spec = pl.BlockSpec((blk, CH), lambda i: (i, 0))
    halves = []
    for sb, c in ((sbA, cA), (sbB, cB)):
        halves.append(pl.pallas_call(
            _fin_ent_body,
            grid=(grid,),
            in_specs=[spec, spec],
            out_specs=spec,
            out_shape=jax.ShapeDtypeStruct((EHALF, CH), _f32),
        )(sb, c))
    return jnp.concatenate(halves, axis=0)[:N_ENTITIES]


def _add_body(a_ref, b_ref, o_ref):
    o_ref[...] = a_ref[...] + b_ref[...]


def _padd(a, b):
    return pl.pallas_call(
        _add_body,
        out_shape=jax.ShapeDtypeStruct(a.shape, a.dtype),
    )(a, b)


def kernel(user_emb, entity_emb, item_emb_cf, relation_weight, edge_index,
           edge_type, inter_row, inter_col):
    head = edge_index[0]
    tail = edge_index[1]
    ktyp = edge_type - 1
    urp, rn16 = _rel_prep(relation_weight)
    urpt = urp.T
    cnt = _count(head)
    cA, cB = cnt[:EACC], cnt[EACC:]

    er, ur_, ir = entity_emb, user_emb, item_emb_cf
    ea, ua, ia = entity_emb, user_emb, item_emb_cf
    for rnd in range(4):
        U, duurf, th1, hn1, nu21 = _ent_prep(ea, urpt)
        sc = _sca(head, tail, ktyp, U, duurf, th1, hn1, nu21, rn16)
        msg = _edge_chain(*sc, ktyp, urp)
        sb = _scb(head, msg)
        ip = _push_item(ua, inter_row, inter_col)
        fusion = _padd(ia, ea[:N_ITEMS])
        up = _push_user(fusion, inter_col, inter_row)
        ea = _fin_entity(sb[:EACC], sb[EACC:], cA, cB)
        ua = _fin_dense(up[:NU_PAD], up[NU_PAD:], N_USERS)
        ia = _fin_dense(ip[:NI_PAD], ip[NI_PAD:], N_ITEMS)
        if rnd == 0 or rnd == 3:
            er = _padd(er, ea)
            ur_ = _padd(ur_, ua)
            ir = _padd(ir, ia)
    return (er, ur_, ir)


# speedup vs baseline: 2.4542x; 1.7165x over previous
import functools

import jax
import jax.numpy as jnp
from jax import lax
from jax.experimental import pallas as pl
from jax.experimental.pallas import tpu as pltpu
from jax.experimental.pallas import tpu_sc as plsc

N_USERS = 8000
N_ITEMS = 4000
N_ENTITIES = 10000
N_EDGES = 320000
N_INTER = 160000
CH = 128

NE_PAD = 10240
NU_PAD = 8192
NI_PAD = 4096

NW = 32
EDGES_PER_W = N_EDGES // NW
SCA_S = 400
SCA_SUB = EDGES_PER_W // SCA_S
SCB_S = 400
_bf16 = jnp.bfloat16
INTER_PER_W = N_INTER // NW
PUSH_S = 200
PUSH_SUB = INTER_PER_W // PUSH_S

_f32 = jnp.float32
_i32 = jnp.int32


def _rel_body(rw_ref, urp_ref, rnb_ref):
    r = rw_ref[...]
    n2 = jnp.sum(r * r, axis=1, keepdims=True)
    rn = jnp.clip(jnp.sqrt(n2), 1e-15, None)
    urp_ref[...] = r / rn
    rnb_ref[...] = jnp.broadcast_to(rn, r.shape)


def _rel_prep(rw):
    rwp = jnp.pad(rw, ((0, 16 - rw.shape[0]), (0, 0)))
    urp, rnb = pl.pallas_call(
        _rel_body,
        out_shape=(jax.ShapeDtypeStruct((16, CH), _f32),
                   jax.ShapeDtypeStruct((16, CH), _f32)),
    )(rwp)
    return urp, rnb[:, 0]


def _idx2_body(h2_ref, o_ref):
    h = h2_ref[...]
    row = lax.broadcasted_iota(_i32, h.shape, 0)
    second = row >= (h.shape[0] // 2)
    v0 = jnp.where(h < EHALF, h, GROW)
    v1 = jnp.where(h >= EHALF, h - EHALF, GROW)
    o_ref[...] = jnp.where(second, v1, v0)


def _idx2_prep(head):
    h2 = jnp.concatenate([head, head]).reshape(2 * N_EDGES // CH, CH)
    out = pl.pallas_call(
        _idx2_body,
        out_shape=jax.ShapeDtypeStruct(h2.shape, _i32),
    )(h2)
    return out.reshape(-1)


def _sca_body(head, tail, E,
              o_eh, o_et,
              idxh, idxt, Eh, Et, sem):
    cid = lax.axis_index("c")
    sid = lax.axis_index("s")
    wid = cid * 16 + sid

    def sub(s, carry):
        base = wid * EDGES_PER_W + s * SCA_S
        sl_out = pl.ds(base, SCA_S)
        c1 = pltpu.async_copy(head.at[sl_out], idxh, sem)
        c2 = pltpu.async_copy(tail.at[sl_out], idxt, sem)
        c1.wait()
        c2.wait()
        g1 = pltpu.async_copy(E.at[idxh], Eh, sem)
        g2 = pltpu.async_copy(E.at[idxt], Et, sem)
        g1.wait()
        g2.wait()
        o1 = pltpu.async_copy(Eh, o_eh.at[sl_out], sem)
        o2 = pltpu.async_copy(Et, o_et.at[sl_out], sem)
        o1.wait()
        o2.wait()
        return carry
    lax.fori_loop(0, SCA_SUB, sub, 0)


def _sca(head, tail, E):
    mesh = plsc.VectorSubcoreMesh(core_axis_name="c", subcore_axis_name="s")
    em = jax.ShapeDtypeStruct((N_EDGES, CH), _f32)
    fn = pl.kernel(
        _sca_body,
        mesh=mesh,
        out_type=(em, em),
        scratch_types=[
            pltpu.VMEM((SCA_S,), _i32), pltpu.VMEM((SCA_S,), _i32),
            pltpu.VMEM((SCA_S, CH), _f32), pltpu.VMEM((SCA_S, CH), _f32),
            pltpu.SemaphoreType.DMA,
        ],
    )
    return fn(head, tail, E)


def _chain_body(eh_ref, et_ref, kt_r, rn_r, urp_ref, msg_ref):
    eh = eh_ref[...]
    et = et_ref[...]
    n2h = jnp.sum(eh * eh, axis=1, keepdims=True)
    hnh = jnp.clip(jnp.sqrt(n2h), 1e-15, None)
    uh = eh / hnh
    g11 = n2h / (hnh * hnh)
    th = jnp.tanh(hnh)
    n2t = jnp.sum(et * et, axis=1, keepdims=True)
    hnt = jnp.clip(jnp.sqrt(n2t), 1e-15, None)
    ut = et / hnt
    g22 = n2t / (hnt * hnt)
    g12 = jnp.sum(uh * ut, axis=1, keepdims=True)

    lane = lax.broadcasted_iota(_i32, (1, 16), 1)
    ohk = (lane == kt_r[...]).astype(_f32)
    urk = jnp.dot(ohk, urp_ref[...], preferred_element_type=_f32)
    g13 = jnp.sum(uh * urk, axis=1, keepdims=True)
    g23 = jnp.sum(ut * urk, axis=1, keepdims=True)
    rnk = jnp.sum(ohk * rn_r[...], axis=1, keepdims=True)

    p2 = th * th * g11
    lam = 2.0 / jnp.clip(1.0 - p2, 1e-15, None)
    a_t = jnp.tanh(lam * hnt * 0.5)
    a_r = jnp.tanh(lam * rnk * 0.5)

    xy1 = th * a_t * g12
    y21 = a_t * a_t * g22
    den1 = jnp.clip(1.0 + 2.0 * xy1 + p2 * y21, 1e-15, None)
    c1 = (1.0 + 2.0 * xy1 + y21) * th / den1
    c2 = (1.0 - p2) * a_t / den1

    xy2 = th * a_r * g13
    y22 = a_r * a_r
    den2 = jnp.clip(1.0 + 2.0 * xy2 + p2 * y22, 1e-15, None)
    d1 = (1.0 + 2.0 * xy2 + y22) * th / den2
    d3 = (1.0 - p2) * a_r / den2

    def quad(v1, v2, v3, w1, w2, w3):
        return (v1 * w1 * g11 + v2 * w2 * g22 + v3 * w3
                + (v1 * w2 + v2 * w1) * g12 + (v1 * w3 + v3 * w1) * g13
                + (v2 * w3 + v3 * w2) * g23)

    xx = quad(c1, c2, 0.0, c1, c2, 0.0)
    yy = quad(d1, 0.0, d3, d1, 0.0, d3)
    xy = quad(c1, c2, 0.0, d1, 0.0, d3)
    den = jnp.clip(1.0 + 2.0 * xy + xx * yy, 1e-15, None)
    A = (1.0 + 2.0 * xy + yy) / den
    B = (1.0 - xx) / den
    e1 = A * c1 + B * d1
    e2 = A * c2
    e3 = B * d3

    pn2 = quad(e1, e2, e3, e1, e2, e3)
    n = jnp.clip(jnp.sqrt(jnp.maximum(pn2, 0.0)), 1e-15, None)
    maxn = 1.0 - 1e-3
    scl = jnp.where(n > maxn, maxn / n, 1.0)
    e1 = e1 * scl
    e2 = e2 * scl
    e3 = e3 * scl
    y2p = pn2 * scl * scl

    xys = -th * (e1 * g11 + e2 * g12 + e3 * g13)
    dens = jnp.clip(1.0 + 2.0 * xys + p2 * y2p, 1e-15, None)
    As = (1.0 + 2.0 * xys + y2p) / dens
    Bs = (1.0 - p2) / dens
    f1 = -As * th + Bs * e1
    f2 = Bs * e2
    f3 = Bs * e3

    sn2 = quad(f1, f2, f3, f1, f2, f3)
    sn = jnp.clip(jnp.sqrt(jnp.maximum(sn2, 0.0)), 1e-15, None)
    snc = jnp.clip(sn, 1e-15, 1.0 - 1e-7)
    atanh = 0.5 * jnp.log((1.0 + snc) / (1.0 - snc))
    fac = (2.0 / lam) * atanh / sn
    msg_ref[...] = fac * (f1 * uh + f2 * ut + f3 * urk)


def _edge_chain(eh, et, ktyp, rn2d, urp):
    blk = 2000
    grid = N_EDGES // blk
    spec = pl.BlockSpec((blk, CH), lambda i: (i, 0))
    spec1 = pl.BlockSpec((blk, 1), lambda i: (i, 0))
    kt2 = ktyp.reshape(N_EDGES, 1)
    msg = pl.pallas_call(
        _chain_body,
        grid=(grid,),
        in_specs=[spec, spec, spec1,
                  pl.BlockSpec((1, 16), lambda i: (0, 0)),
                  pl.BlockSpec((16, CH), lambda i: (0, 0))],
        out_specs=spec,
        out_shape=jax.ShapeDtypeStruct((N_EDGES, CH), _f32),
    )(eh, et, kt2, rn2d, urp)
    return msg


EHALF = 5120
EACC = 5376
GROW = EHALF
SCB_PER_SC_W = N_EDGES // 16
SCB_SUB = SCB_PER_SC_W // SCA_S


def _scb_body(idx2flat, msg, sb, idx2, rows, zb, acc_sp, sem):
    cid = lax.axis_index("c")
    sid = lax.axis_index("s")
    stripe = EACC // 16

    for r in range(16):
        for c in range(8):
            zb[r, pl.ds(c * 16, 16)] = jnp.zeros((16,), _f32)

    def zfill(z, carry):
        pltpu.sync_copy(zb, acc_sp.at[pl.ds(sid * stripe + z * 16, 16)])
        return carry
    lax.fori_loop(0, stripe // 16, zfill, 0)
    plsc.subcore_barrier()

    def sub(s, carry):
        base = sid * SCB_PER_SC_W + s * SCB_S
        c1 = pltpu.async_copy(
            idx2flat.at[pl.ds(cid * N_EDGES + base, SCB_S)], idx2, sem)
        c2 = pltpu.async_copy(msg.at[pl.ds(base, SCB_S)], rows, sem)
        c1.wait()
        c2.wait()
        pltpu.sync_copy(rows, acc_sp.at[idx2], add=True)
        return carry
    lax.fori_loop(0, SCB_PER_SC_W // SCB_S, sub, 0)
    plsc.subcore_barrier()

    pltpu.sync_copy(acc_sp.at[pl.ds(sid * stripe, stripe)],
                    sb.at[pl.ds(cid * EACC + sid * stripe, stripe)])


def _scb(idx2flat, msg):
    mesh = plsc.VectorSubcoreMesh(core_axis_name="c", subcore_axis_name="s")
    fn = pl.kernel(
        _scb_body,
        mesh=mesh,
        out_type=jax.ShapeDtypeStruct((2 * EACC, CH), _f32),
        scratch_types=[
            pltpu.VMEM((SCB_S,), _i32),
            pltpu.VMEM((SCB_S, CH), _f32),
            pltpu.VMEM((16, CH), _f32),
            pltpu.VMEM_SHARED((EACC, CH), _f32),
            pltpu.SemaphoreType.DMA,
        ],
    )
    return fn(idx2flat, msg)


_CNT_S = 80


def _cnt_body(idx2flat, cc, idx2, ones, zb, acc_sp, sem):
    cid = lax.axis_index("c")
    sid = lax.axis_index("s")
    stripe = EACC // 16

    for r in range(16):
        for c in range(8):
            zb[r, pl.ds(c * 16, 16)] = jnp.zeros((16,), _f32)
    for r in range(_CNT_S):
        for c in range(8):
            ones[r, pl.ds(c * 16, 16)] = jnp.ones((16,), _f32)

    def zfill(z, carry):
        pltpu.sync_copy(zb, acc_sp.at[pl.ds(sid * stripe + z * 16, 16)])
        return carry
    lax.fori_loop(0, stripe // 16, zfill, 0)
    plsc.subcore_barrier()

    def sub(s, carry):
        base = sid * SCB_PER_SC_W + s * _CNT_S
        pltpu.sync_copy(idx2flat.at[pl.ds(cid * N_EDGES + base, _CNT_S)], idx2)
        pltpu.sync_copy(ones, acc_sp.at[idx2], add=True)
        return carry
    lax.fori_loop(0, SCB_PER_SC_W // _CNT_S, sub, 0)
    plsc.subcore_barrier()

    pltpu.sync_copy(acc_sp.at[pl.ds(sid * stripe, stripe)],
                    cc.at[pl.ds(cid * EACC + sid * stripe, stripe)])


def _count(idx2flat):
    mesh = plsc.VectorSubcoreMesh(core_axis_name="c", subcore_axis_name="s")
    fn = pl.kernel(
        _cnt_body,
        mesh=mesh,
        out_type=jax.ShapeDtypeStruct((2 * EACC, CH), _f32),
        scratch_types=[
            pltpu.VMEM((_CNT_S,), _i32),
            pltpu.VMEM((_CNT_S, CH), _f32),
            pltpu.VMEM((16, CH), _f32),
            pltpu.VMEM_SHARED((EACC, CH), _f32),
            pltpu.SemaphoreType.DMA,
        ],
    )
    return fn(idx2flat)


def _make_push(n_dst_pad):
    stripe = n_dst_pad // 16

    def body(table, src_idx, dst_idx, po,
             isrc, idst, rows, zb, acc_sp, sem):
        cid = lax.axis_index("c")
        sid = lax.axis_index("s")
        wid = cid * 16 + sid

        for r in range(16):
            for c in range(8):
                zb[r, pl.ds(c * 16, 16)] = jnp.zeros((16,), _f32)

        def zfill(z, carry):
            pltpu.sync_copy(zb, acc_sp.at[pl.ds(sid * stripe + z * 16, 16)])
            return carry
        lax.fori_loop(0, stripe // 16, zfill, 0)
        plsc.subcore_barrier()

        def sub(s, carry):
            base = wid * INTER_PER_W + s * PUSH_S
            c1 = pltpu.async_copy(src_idx.at[pl.ds(base, PUSH_S)], isrc, sem)
            c2 = pltpu.async_copy(dst_idx.at[pl.ds(base, PUSH_S)], idst, sem)
            c1.wait()
            c2.wait()
            pltpu.async_copy(table.at[isrc], rows, sem).wait()
            pltpu.sync_copy(rows, acc_sp.at[idst], add=True)
            return carry
        lax.fori_loop(0, PUSH_SUB, sub, 0)
        plsc.subcore_barrier()

        pltpu.sync_copy(acc_sp.at[pl.ds(sid * stripe, stripe)],
                        po.at[pl.ds(cid * n_dst_pad + sid * stripe, stripe)])

    mesh = plsc.VectorSubcoreMesh(core_axis_name="c", subcore_axis_name="s")
    fn = pl.kernel(
        body,
        mesh=mesh,
        out_type=jax.ShapeDtypeStruct((2 * n_dst_pad, CH), _f32),
        scratch_types=[
            pltpu.VMEM((PUSH_S,), _i32), pltpu.VMEM((PUSH_S,), _i32),
            pltpu.VMEM((PUSH_S, CH), _f32),
            pltpu.VMEM((16, CH), _f32),
            pltpu.VMEM_SHARED((n_dst_pad, CH), _f32),
            pltpu.SemaphoreType.DMA,
        ],
    )
    return fn


_make_push = functools.cache(_make_push)


def _push_item(table, src_idx, dst_idx):
    return _make_push(NI_PAD)(table, src_idx, dst_idx)


def _push_user(table, src_idx, dst_idx):
    return _make_push(NU_PAD)(table, src_idx, dst_idx)


def _dr3(x, eps=1e-12):
    out = None
    for _ in range(3):
        n = jnp.clip(jnp.sqrt(jnp.sum(x * x, axis=1, keepdims=True)), eps, None)
        x = x / n
        out = x if out is None else out + x
    return out


def _fin_dense_body(p0_ref, p1_ref, o_ref):
    o_ref[...] = _dr3(p0_ref[...] + p1_ref[...])


def _fin_dense(p0, p1, n):
    blk = 1000
    grid = n // blk
    spec = pl.BlockSpec((blk, CH), lambda i: (i, 0))
    return pl.pallas_call(
        _fin_dense_body,
        grid=(grid,),
        in_specs=[spec, spec],
        out_specs=spec,
        out_shape=jax.ShapeDtypeStruct((n, CH), _f32),
    )(p0, p1)


def _fin_ent_body(sb_ref, c_ref, o_ref):
    cnt = jnp.clip(c_ref[...], 1.0, None)
    o_ref[...] = _dr3(sb_ref[...] / cnt)


def _fin_entity(sbA, sbB, cA, cB):
    blk = 1024
    grid = EHALF // blk
    spec = pl.BlockSpec((blk, CH), lambda i: (i, 0))
    halves = []
    for sb, c in ((sbA, cA), (sbB, cB)):
        halves.append(pl.pallas_call(
            _fin_ent_body,
            grid=(grid,),
            in_specs=[spec, spec],
            out_specs=spec,
            out_shape=jax.ShapeDtypeStruct((EHALF, CH), _f32),
        )(sb, c))
    return jnp.concatenate(halves, axis=0)[:N_ENTITIES]


def _add_body(a_ref, b_ref, o_ref):
    o_ref[...] = a_ref[...] + b_ref[...]


def _padd(a, b):
    return pl.pallas_call(
        _add_body,
        out_shape=jax.ShapeDtypeStruct(a.shape, a.dtype),
    )(a, b)


def kernel(user_emb, entity_emb, item_emb_cf, relation_weight, edge_index,
           edge_type, inter_row, inter_col):
    head = edge_index[0]
    tail = edge_index[1]
    ktyp = edge_type - 1
    urp, rn16 = _rel_prep(relation_weight)
    rn2d = rn16.reshape(1, 16)
    idx2flat = _idx2_prep(head)
    cnt = _count(idx2flat)
    cA, cB = cnt[:EACC], cnt[EACC:]

    er, ur_, ir = entity_emb, user_emb, item_emb_cf
    ea, ua, ia = entity_emb, user_emb, item_emb_cf
    for rnd in range(4):
        eh, et = _sca(head, tail, ea)
        msg = _edge_chain(eh, et, ktyp, rn2d, urp)
        sb = _scb(idx2flat, msg)
        ip = _push_item(ua, inter_row, inter_col)
        fusion = _padd(ia, ea[:N_ITEMS])
        up = _push_user(fusion, inter_col, inter_row)
        ea = _fin_entity(sb[:EACC], sb[EACC:], cA, cB)
        ua = _fin_dense(up[:NU_PAD], up[NU_PAD:], N_USERS)
        ia = _fin_dense(ip[:NI_PAD], ip[NI_PAD:], N_ITEMS)
        if rnd == 0 or rnd == 3:
            er = _padd(er, ea)
            ur_ = _padd(ur_, ua)
            ir = _padd(ir, ia)
    return (er, ur_, ir)
